# baseline (speedup 1.0000x reference)
import functools

import jax
import jax.numpy as jnp
from jax import lax
from jax.experimental import pallas as pl
from jax.experimental.pallas import tpu as pltpu
from jax.experimental.pallas import tpu_sc as plsc

_NUM_CORES = 2
_NUM_SUBCORES = 16
_NUM_WORKERS = _NUM_CORES * _NUM_SUBCORES

_CHUNK = 128


@functools.cache
def _build(batch: int, dim: int):
    b_per_w = batch // _NUM_WORKERS
    n_chunks = b_per_w // _CHUNK
    mesh = plsc.VectorSubcoreMesh(core_axis_name="c", subcore_axis_name="s")

    @functools.partial(
        pl.kernel,
        mesh=mesh,
        out_type=jax.ShapeDtypeStruct((batch, dim), jnp.float32),
        scratch_types=[
            pltpu.VMEM((n_chunks, _CHUNK), jnp.int32),
            pltpu.VMEM((b_per_w, dim), jnp.float32),
            pltpu.SemaphoreType.DMA,
        ],
        compiler_params=pltpu.CompilerParams(use_tc_tiling_on_sc=False),
    )
    def gather_kernel(ids_hbm, table_hbm, out_hbm, idx_v, rows_v, sem):
        wid = lax.axis_index("s") * _NUM_CORES + lax.axis_index("c")
        base = wid * b_per_w
        pltpu.sync_copy(ids_hbm.at[wid], idx_v)
        copies = [
            pltpu.async_copy(
                table_hbm.at[idx_v.at[j]],
                rows_v.at[pl.ds(j * _CHUNK, _CHUNK)],
                sem,
            )
            for j in range(n_chunks)
        ]
        for c in copies:
            c.wait()
        pltpu.sync_copy(rows_v, out_hbm.at[pl.ds(base, b_per_w)])

    return gather_kernel


def kernel(ids, table):
    batch = ids.shape[0]
    dim = table.shape[1]
    ids_r = ids.astype(jnp.int32).reshape(
        _NUM_WORKERS, batch // _NUM_WORKERS // _CHUNK, _CHUNK
    )
    return _build(batch, dim)(ids_r, table)


# speedup vs baseline: 2.2023x; 2.2023x over previous
import functools

import jax
import jax.numpy as jnp
from jax import lax
from jax.experimental import pallas as pl
from jax.experimental.pallas import tpu as pltpu
from jax.experimental.pallas import tpu_sc as plsc

_NUM_CORES = 2
_NUM_SUBCORES = 16
_NUM_WORKERS = _NUM_CORES * _NUM_SUBCORES
_LANES = 16

_CH = 512
_CH_SHIFT = 9
_NSTAGE = 8
_INTERPRET = False


@functools.cache
def _build(batch: int, dim: int, pool: int, interpret=False):
    n_full = pool // _CH
    tail = pool - n_full * _CH
    base_per_w = n_full // _NUM_WORKERS
    extra = n_full - base_per_w * _NUM_WORKERS
    id_pieces = batch // 1024
    out_rows = batch + _LANES
    bshift = (batch - 1).bit_length()
    dump0 = batch + _LANES
    mesh = plsc.VectorSubcoreMesh(
        core_axis_name="c",
        subcore_axis_name="s",
        num_cores=_NUM_CORES,
        num_subcores=_NUM_SUBCORES,
    )

    @functools.partial(
        pl.kernel,
        mesh=mesh,
        out_type=jax.ShapeDtypeStruct((out_rows, 2 * dim), jnp.float32),
        scratch_types=[
            pltpu.VMEM((1024,), jnp.int32),
            pltpu.VMEM((batch + 2 * _LANES,), jnp.int32),
            pltpu.VMEM((batch + 2 * _LANES,), jnp.int32),
            pltpu.VMEM((2, dim, _CH), jnp.float32),
            pltpu.VMEM((dim, tail), jnp.float32),
            pltpu.VMEM((_NSTAGE, _LANES, 2 * dim), jnp.float32),
            pltpu.VMEM((_NSTAGE, _LANES), jnp.int32),
            pltpu.SemaphoreType.DMA,
            pltpu.SemaphoreType.DMA,
            pltpu.SemaphoreType.DMA,
        ],
        compiler_params=pltpu.CompilerParams(needs_layout_passes=False),
        interpret=interpret,
    )
    def gather_kernel(
        ids_hbm,
        tableT_hbm,
        tailT_hbm,
        outp_hbm,
        idp_v,
        hit_v,
        chit_v,
        chunk_v,
        tail_v,
        stage_v,
        sidx_v,
        sem_c0,
        sem_c1,
        sem_s,
    ):
        wid = lax.axis_index("s") * _NUM_CORES + lax.axis_index("c")
        is_last = wid == _NUM_WORKERS - 1
        my_n = jnp.where(wid < extra, base_per_w + 1, base_per_w)
        cg0 = jnp.where(
            wid < extra,
            wid * (base_per_w + 1),
            extra * (base_per_w + 1) + (wid - extra) * base_per_w,
        )
        lo = cg0 * _CH
        hi = jnp.where(is_last, pool, (cg0 + my_n) * _CH)
        lanes = lax.iota(jnp.int32, _LANES)

        def scan_piece(p, ptr):
            pltpu.sync_copy(ids_hbm.at[pl.ds(p * 1024, 1024)], idp_v)

            def scan_vreg(k, ptr):
                v = idp_v[pl.ds(k * _LANES, _LANES)]
                b = p * 1024 + k * _LANES + lanes
                m = (v >= lo) & (v < hi)
                mi = m.astype(jnp.int32)
                packed = lax.shift_left(v - lo, bshift) | b
                pos = jnp.where(m, ptr + jnp.cumsum(mi) - 1, dump0 + lanes)
                plsc.store_scatter(hit_v, [pos], packed)
                return ptr + jnp.sum(mi)

            return lax.fori_loop(0, 1024 // _LANES, scan_vreg, ptr)

        n_w = lax.fori_loop(0, id_pieces, scan_piece, jnp.int32(0))
        hpad = (n_w // _LANES) * _LANES
        hv = hit_v[pl.ds(hpad, _LANES)]
        hit_v[pl.ds(hpad, _LANES)] = jnp.where(hpad + lanes >= n_w, -1, hv)
        n_hgroups = (n_w + _LANES - 1) // _LANES

        sems = (sem_c0, sem_c1)

        def chunk_start(c, buf):
            pltpu.async_copy(
                tableT_hbm.at[:, pl.ds((cg0 + c) * _CH, _CH)],
                chunk_v.at[buf],
                sems[buf],
            )

        def chunk_wait(buf):
            pltpu.make_async_copy(
                tableT_hbm.at[:, pl.ds(0, _CH)], chunk_v.at[buf], sems[buf]
            ).wait()

        @pl.when(my_n > 0)
        def _():
            chunk_start(jnp.int32(0), 0)

        @pl.when(is_last)
        def _():
            pltpu.sync_copy(tailT_hbm, tail_v)

        def compact_chunk(c):
            def compact(k, cptr):
                v = hit_v[pl.ds(k * _LANES, _LANES)]
                m = lax.shift_right_logical(v, _CH_SHIFT + bshift) == c
                mi = m.astype(jnp.int32)
                pos = jnp.where(m, cptr + jnp.cumsum(mi) - 1, dump0 + lanes)
                plsc.store_scatter(chit_v, [pos], v)
                return cptr + jnp.sum(mi)

            return lax.fori_loop(0, n_hgroups, compact, jnp.int32(0))

        def extract_groups(n_c, g_tot, from_tail, buf):
            n_groups = (n_c + _LANES - 1) // _LANES

            def per_group(g, g_tot):
                sbuf = lax.rem(g_tot, _NSTAGE)

                @pl.when((g_tot > 0) & (sbuf == 0))
                def _():
                    def drain_all(i, _):
                        pltpu.make_async_copy(
                            stage_v.at[0], outp_hbm.at[sidx_v.at[0]], sem_s
                        ).wait()
                        return _

                    lax.fori_loop(0, _NSTAGE, drain_all, jnp.int32(0))

                packed = chit_v[pl.ds(g * _LANES, _LANES)]
                cols = lax.shift_right_logical(packed, bshift) & (_CH - 1)
                bpos = packed & (batch - 1)
                sidx_v[sbuf] = jnp.where(
                    g * _LANES + lanes < n_c, bpos, jnp.int32(batch) + lanes
                )

                def per_word(j, _):
                    if from_tail:
                        w = plsc.load_gather(tail_v, [jnp.full((_LANES,), j), cols])
                    else:
                        w = plsc.load_gather(
                            chunk_v.at[buf], [jnp.full((_LANES,), j), cols]
                        )
                    plsc.store_scatter(
                        stage_v,
                        [jnp.full((_LANES,), sbuf), lanes, jnp.full((_LANES,), j)],
                        w,
                    )
                    return _

                lax.fori_loop(0, dim, per_word, jnp.int32(0))
                pltpu.async_copy(
                    stage_v.at[sbuf], outp_hbm.at[sidx_v.at[sbuf]], sem_s
                )
                return g_tot + 1

            return lax.fori_loop(0, n_groups, per_group, g_tot)

        max_chunks = base_per_w + (1 if extra else 0)
        n_pairs = (max_chunks + 1) // 2

        def chunk_step(c, g_tot, buf):
            @pl.when(c + 1 < my_n)
            def _():
                chunk_start(c + 1, 1 - buf)

            @pl.when(c < my_n)
            def _():
                chunk_wait(buf)

            n_c = compact_chunk(c)
            return extract_groups(n_c, g_tot, False, buf)

        def per_pair(p, g_tot):
            g_tot = chunk_step(2 * p, g_tot, 0)
            return chunk_step(2 * p + 1, g_tot, 1)

        g_tot = lax.fori_loop(0, n_pairs, per_pair, jnp.int32(0))

        @pl.when(is_last)
        def _():
            n_c = compact_chunk(my_n)
            g2 = extract_groups(n_c, g_tot, True, 0)

            rem = jnp.where(g2 > 0, g2 - _NSTAGE * ((g2 - 1) // _NSTAGE), 0)

            def drain(i, _):
                pltpu.make_async_copy(
                    stage_v.at[0], outp_hbm.at[sidx_v.at[0]], sem_s
                ).wait()
                return _

            lax.fori_loop(0, rem, drain, jnp.int32(0))

        @pl.when(jnp.logical_not(is_last))
        def _():
            rem = jnp.where(g_tot > 0, g_tot - _NSTAGE * ((g_tot - 1) // _NSTAGE), 0)

            def drain(i, _):
                pltpu.make_async_copy(
                    stage_v.at[0], outp_hbm.at[sidx_v.at[0]], sem_s
                ).wait()
                return _

            lax.fori_loop(0, rem, drain, jnp.int32(0))

    return gather_kernel


def kernel(ids, table):
    batch = ids.shape[0]
    pool, dim = table.shape
    n_full = pool // _CH
    ids32 = ids.astype(jnp.int32)
    tableT = table.T
    tailT = tableT[:, n_full * _CH :]
    outp = _build(batch, dim, pool, _INTERPRET)(ids32, tableT, tailT)
    return outp[:batch, :dim]
